# Initial kernel scaffold; baseline (speedup 1.0000x reference)
import functools

import jax
import jax.numpy as jnp
from jax import lax
from jax.experimental import pallas as pl
from jax.experimental.pallas import tpu as pltpu
from jax.experimental.pallas import tpu_sc as plsc

AN = 5
D_MODEL = 64
NHEAD = 8
DPH = D_MODEL // NHEAD
DFF = 256
N = 10000
NB = 16
DA = D_MODEL * AN
PAIRS = N * NB

NWORK = 32
PER_W = PAIRS // NWORK
WIN = 40
NWIN = PER_W // WIN

BA = 1000
BC = 200


def _proj_body(tgt_ref, wk_ref, wq_ref, tk_ref, tv_ref):
    t = tgt_ref[...]
    tk_ref[...] = jnp.dot(t, wk_ref[...], preferred_element_type=jnp.float32)
    tv_ref[...] = jnp.dot(t, wq_ref[...], preferred_element_type=jnp.float32)


def _project(tgt, wkt, wqt):
    grid = (N // BA,)
    return pl.pallas_call(
        _proj_body,
        grid=grid,
        in_specs=[
            pl.BlockSpec((BA, DA), lambda i: (i, 0)),
            pl.BlockSpec((DA, DA), lambda i: (0, 0)),
            pl.BlockSpec((DA, DA), lambda i: (0, 0)),
        ],
        out_specs=[
            pl.BlockSpec((BA, DA), lambda i: (i, 0)),
            pl.BlockSpec((BA, DA), lambda i: (i, 0)),
        ],
        out_shape=[
            jax.ShapeDtypeStruct((N, DA), jnp.float32),
            jax.ShapeDtypeStruct((N, DA), jnp.float32),
        ],
    )(tgt, wkt, wqt)


def _gather(tk, tv, idx_t):
    mesh = plsc.VectorSubcoreMesh(core_axis_name="c", subcore_axis_name="s")

    @functools.partial(
        pl.kernel,
        out_type=(
            jax.ShapeDtypeStruct((PAIRS, DA), jnp.float32),
            jax.ShapeDtypeStruct((PAIRS, DA), jnp.float32),
        ),
        mesh=mesh,
        scratch_types=[
            pltpu.VMEM((PER_W,), jnp.int32),
            pltpu.VMEM((WIN, DA), jnp.float32),
            pltpu.VMEM((WIN, DA), jnp.float32),
            pltpu.SemaphoreType.DMA,
            pltpu.SemaphoreType.DMA,
        ],
    )
    def k(tk_hbm, tv_hbm, idx_hbm, gk_hbm, gv_hbm, idx_v, bk, bv, sk, sv):
        wid = lax.axis_index("s") * 2 + lax.axis_index("c")
        base = pl.multiple_of(wid * PER_W, 8)
        pltpu.sync_copy(idx_hbm.at[pl.ds(base, PER_W)], idx_v)

        @pl.loop(0, NWIN)
        def _(w):
            off = pl.multiple_of(w * WIN, 8)
            isl = idx_v.at[pl.ds(off, WIN)]
            ck = pltpu.async_copy(tk_hbm.at[isl], bk, sk)
            cv = pltpu.async_copy(tv_hbm.at[isl], bv, sv)
            ck.wait()
            pltpu.sync_copy(bk, gk_hbm.at[pl.ds(base + off, WIN)])
            cv.wait()
            pltpu.sync_copy(bv, gv_hbm.at[pl.ds(base + off, WIN)])

    return k(tk, tv, idx_t)


def _attn_body(tgt_ref, tv_ref, gk_ref, gv_ref, dist_ref, p_ref, h8_ref,
               h8t_ref, wo_ref, ma_ref, w1_ref, b1_ref, w2_ref, b2_ref,
               g1_ref, be1_ref, g2p_ref, be2d_ref, out_ref):
    scale = 1.0 / (DPH ** 0.5)
    q = tv_ref[...]
    h8 = h8_ref[...]
    logits = []
    for n in range(NB):
        prod = q * gk_ref[n]
        dn = jnp.dot(prod, h8, preferred_element_type=jnp.float32)
        logits.append(dn + dist_ref[n] * scale)
    m = logits[0]
    for n in range(1, NB):
        m = jnp.maximum(m, logits[n])
    es = [jnp.exp(l - m) for l in logits]
    s = es[0]
    for n in range(1, NB):
        s = s + es[n]
    inv = 1.0 / s
    h8t = h8t_ref[...]
    acc = (jnp.dot(es[0] * inv, h8t, preferred_element_type=jnp.float32)
           * gv_ref[0])
    for n in range(1, NB):
        w = es[n] * inv
        acc = acc + jnp.dot(w, h8t, preferred_element_type=jnp.float32) * gv_ref[n]

    tp = jnp.dot(tgt_ref[...], p_ref[...], preferred_element_type=jnp.float32)
    x = tp + jnp.dot(acc, wo_ref[...], preferred_element_type=jnp.float32)
    ma = ma_ref[...]
    mb = jnp.dot(x, ma, preferred_element_type=jnp.float32)
    xc = x - mb
    vb = jnp.dot(xc * xc, ma, preferred_element_type=jnp.float32)
    xn = xc * lax.rsqrt(vb + 1e-5) * g1_ref[...] + be1_ref[...]
    h = jnp.maximum(
        jnp.dot(xn, w1_ref[...], preferred_element_type=jnp.float32)
        + b1_ref[...], 0.0)
    x2 = xn + jnp.dot(h, w2_ref[...], preferred_element_type=jnp.float32) + b2_ref[...]
    mb2 = jnp.dot(x2, ma, preferred_element_type=jnp.float32)
    xc2 = x2 - mb2
    vb2 = jnp.dot(xc2 * xc2, ma, preferred_element_type=jnp.float32)
    out_ref[...] = (jnp.dot(xc2 * lax.rsqrt(vb2 + 1e-5), g2p_ref[...],
                            preferred_element_type=jnp.float32)
                    + be2d_ref[...])


def _attn_tail(tgt, tv, gk3, gv3, dist_t, p_mat, h8, h8t, wo, ma, w1k, b1k,
               w2k, b2k, g1t, be1t, g2p, be2d):
    grid = (N // BC,)
    full = lambda r, c: pl.BlockSpec((r, c), lambda i: (0, 0))
    return pl.pallas_call(
        _attn_body,
        grid=grid,
        in_specs=[
            pl.BlockSpec((BC, DA), lambda i: (i, 0)),
            pl.BlockSpec((BC, DA), lambda i: (i, 0)),
            pl.BlockSpec((NB, BC, DA), lambda i: (0, i, 0)),
            pl.BlockSpec((NB, BC, DA), lambda i: (0, i, 0)),
            pl.BlockSpec((NB, BC, NHEAD), lambda i: (0, i, 0)),
            full(DA, DA),
            full(DA, NHEAD),
            full(NHEAD, DA),
            full(DA, DA),
            full(DA, DA),
            full(DA, AN * DFF),
            full(1, AN * DFF),
            full(AN * DFF, DA),
            full(1, DA),
            full(1, DA),
            full(1, DA),
            full(DA, DA),
            full(1, DA),
        ],
        out_specs=pl.BlockSpec((BC, DA), lambda i: (i, 0)),
        out_shape=jax.ShapeDtypeStruct((N, DA), jnp.float32),
    )(tgt, tv, gk3, gv3, dist_t, p_mat, h8, h8t, wo, ma, w1k, b1k, w2k, b2k,
      g1t, be1t, g2p, be2d)


def kernel(tgt, index_pair, cnt, sh, dist_atten, Wq, Wk, Wout, W1, b1, W2,
           b2, g1, be1, g2, be2):
    del cnt, sh
    f32 = jnp.float32
    eye5 = jnp.eye(AN, dtype=f32)

    def expand_in(w):
        m = w.T[:, None, None, :] * eye5[None, :, :, None]
        return m.reshape(DA, DA)

    wkt = expand_in(Wk)
    wqt = expand_in(Wq)
    p_mat = expand_in(jnp.eye(D_MODEL, dtype=f32))
    wo = jnp.kron(eye5, Wout.T)
    ma = jnp.kron(eye5, jnp.full((D_MODEL, D_MODEL), 1.0 / D_MODEL, f32))
    w1k = jnp.kron(eye5, W1.T)
    w2k = jnp.kron(eye5, W2.T)
    b1k = jnp.tile(b1, AN)[None, :]
    b2k = jnp.tile(b2, AN)[None, :]
    g1t = jnp.tile(g1, AN)[None, :]
    be1t = jnp.tile(be1, AN)[None, :]
    g2t = jnp.tile(g2, AN)
    be2t = jnp.tile(be2, AN)
    scale = 1.0 / (DPH ** 0.5)
    dmaj = jnp.tile(jnp.arange(D_MODEL), AN)
    h8 = (dmaj[:, None] // DPH == jnp.arange(NHEAD)[None, :]).astype(f32)
    h8s = h8 * scale
    h8t = h8.T
    g2p = g2t[:, None] * p_mat.T
    be2d = be2t[None, :] @ p_mat.T

    tk, tv = _project(tgt, wkt, wqt)
    idx_t = index_pair.astype(jnp.int32).T.reshape(-1)
    gk, gv = _gather(tk, tv, idx_t)
    gk3 = gk.reshape(NB, N, DA)
    gv3 = gv.reshape(NB, N, DA)
    dist_t = jnp.transpose(dist_atten, (1, 0, 2))
    return _attn_tail(tgt, tv, gk3, gv3, dist_t, p_mat, h8s, h8t, wo, ma,
                      w1k, b1k, w2k, b2k, g1t, be1t, g2p, be2d)


# speedup vs baseline: 7.6723x; 7.6723x over previous
import functools

import jax
import jax.numpy as jnp
from jax import lax
from jax.experimental import pallas as pl
from jax.experimental.pallas import tpu as pltpu
from jax.experimental.pallas import tpu_sc as plsc

AN = 5
D_MODEL = 64
NHEAD = 8
DPH = D_MODEL // NHEAD
DFF = 256
N = 10000
NB = 16
DA = D_MODEL * AN
DKV = 2 * DA
PAIRS = N * NB

NWORK = 32
PER_W = PAIRS // NWORK
WIN = 40
NWIN = PER_W // WIN

BA = 1000
BC = 200


def _proj_body(tgt_ref, wkv_ref, wqq_ref, kv_ref, tq_ref):
    t = tgt_ref[...]
    kv_ref[...] = jnp.dot(t, wkv_ref[...], preferred_element_type=jnp.float32)
    tq_ref[...] = jnp.dot(t, wqq_ref[...], preferred_element_type=jnp.float32)


def _project(tgt, wkv, wqq):
    grid = (N // BA,)
    return pl.pallas_call(
        _proj_body,
        grid=grid,
        in_specs=[
            pl.BlockSpec((BA, DA), lambda i: (i, 0)),
            pl.BlockSpec((DA, DKV), lambda i: (0, 0)),
            pl.BlockSpec((DA, DKV), lambda i: (0, 0)),
        ],
        out_specs=[
            pl.BlockSpec((BA, DKV), lambda i: (i, 0)),
            pl.BlockSpec((BA, DKV), lambda i: (i, 0)),
        ],
        out_shape=[
            jax.ShapeDtypeStruct((N, DKV), jnp.float32),
            jax.ShapeDtypeStruct((N, DKV), jnp.float32),
        ],
    )(tgt, wkv, wqq)


def _gather(kv, idx_t):
    mesh = plsc.VectorSubcoreMesh(core_axis_name="c", subcore_axis_name="s")

    @functools.partial(
        pl.kernel,
        out_type=jax.ShapeDtypeStruct((PAIRS, DKV), jnp.float32),
        mesh=mesh,
        scratch_types=[
            pltpu.VMEM((PER_W,), jnp.int32),
            pltpu.VMEM((WIN, DKV), jnp.float32),
            pltpu.VMEM((WIN, DKV), jnp.float32),
            pltpu.SemaphoreType.DMA,
            pltpu.SemaphoreType.DMA,
        ],
    )
    def k(kv_hbm, idx_hbm, g_hbm, idx_v, b0, b1, s0, s1):
        wid = lax.axis_index("s") * 2 + lax.axis_index("c")
        base = pl.multiple_of(wid * PER_W, 8)
        pltpu.sync_copy(idx_hbm.at[pl.ds(base, PER_W)], idx_v)

        bufs = (b0, b1)
        sems = (s0, s1)
        pltpu.async_copy(kv_hbm.at[idx_v.at[pl.ds(0, WIN)]], b0, s0)

        @pl.loop(0, NWIN)
        def _(w):
            nxt = w + 1

            @pl.when(nxt < NWIN)
            def _():
                off_n = pl.multiple_of(nxt * WIN, 8)

                @pl.when(lax.rem(nxt, 2) == 0)
                def _():
                    pltpu.async_copy(
                        kv_hbm.at[idx_v.at[pl.ds(off_n, WIN)]], bufs[0], sems[0])

                @pl.when(lax.rem(nxt, 2) == 1)
                def _():
                    pltpu.async_copy(
                        kv_hbm.at[idx_v.at[pl.ds(off_n, WIN)]], bufs[1], sems[1])

            off = pl.multiple_of(w * WIN, 8)

            @pl.when(lax.rem(w, 2) == 0)
            def _():
                pltpu.make_async_copy(
                    kv_hbm.at[pl.ds(0, WIN)], bufs[0], sems[0]).wait()
                pltpu.sync_copy(bufs[0], g_hbm.at[pl.ds(base + off, WIN)])

            @pl.when(lax.rem(w, 2) == 1)
            def _():
                pltpu.make_async_copy(
                    kv_hbm.at[pl.ds(0, WIN)], bufs[1], sems[1]).wait()
                pltpu.sync_copy(bufs[1], g_hbm.at[pl.ds(base + off, WIN)])

    return k(kv, idx_t)


def _attn_body(tgt_ref, tq_ref, g_ref, dist_ref, p_ref, h8_ref,
               h8t_ref, wo_ref, ma_ref, w1_ref, b1_ref, w2_ref, b2_ref,
               g1_ref, be1_ref, g2p_ref, be2d_ref, out_ref):
    scale = 1.0 / (DPH ** 0.5)
    q = tq_ref[...]
    h8 = h8_ref[...]
    logits = []
    for n in range(NB):
        prod = q * g_ref[n]
        dn = jnp.dot(prod, h8, preferred_element_type=jnp.float32)
        logits.append(dn + dist_ref[n] * scale)
    m = logits[0]
    for n in range(1, NB):
        m = jnp.maximum(m, logits[n])
    es = [jnp.exp(l - m) for l in logits]
    s = es[0]
    for n in range(1, NB):
        s = s + es[n]
    inv = 1.0 / s
    h8t = h8t_ref[...]
    acc = (jnp.dot(es[0] * inv, h8t, preferred_element_type=jnp.float32)
           * g_ref[0])
    for n in range(1, NB):
        w = es[n] * inv
        acc = acc + jnp.dot(w, h8t, preferred_element_type=jnp.float32) * g_ref[n]

    tp = jnp.dot(tgt_ref[...], p_ref[...], preferred_element_type=jnp.float32)
    x = tp + jnp.dot(acc, wo_ref[...], preferred_element_type=jnp.float32)
    ma = ma_ref[...]
    mb = jnp.dot(x, ma, preferred_element_type=jnp.float32)
    xc = x - mb
    vb = jnp.dot(xc * xc, ma, preferred_element_type=jnp.float32)
    xn = xc * lax.rsqrt(vb + 1e-5) * g1_ref[...] + be1_ref[...]
    h = jnp.maximum(
        jnp.dot(xn, w1_ref[...], preferred_element_type=jnp.float32)
        + b1_ref[...], 0.0)
    x2 = xn + jnp.dot(h, w2_ref[...], preferred_element_type=jnp.float32) + b2_ref[...]
    mb2 = jnp.dot(x2, ma, preferred_element_type=jnp.float32)
    xc2 = x2 - mb2
    vb2 = jnp.dot(xc2 * xc2, ma, preferred_element_type=jnp.float32)
    out_ref[...] = (jnp.dot(xc2 * lax.rsqrt(vb2 + 1e-5), g2p_ref[...],
                            preferred_element_type=jnp.float32)
                    + be2d_ref[...])


def _attn_tail(tgt, tq, g3, dist_t, p_mat, h8, h8t, wo, ma, w1k, b1k,
               w2k, b2k, g1t, be1t, g2p, be2d):
    grid = (N // BC,)
    full = lambda r, c: pl.BlockSpec((r, c), lambda i: (0, 0))
    return pl.pallas_call(
        _attn_body,
        grid=grid,
        in_specs=[
            pl.BlockSpec((BC, DA), lambda i: (i, 0)),
            pl.BlockSpec((BC, DKV), lambda i: (i, 0)),
            pl.BlockSpec((NB, BC, DKV), lambda i: (0, i, 0)),
            pl.BlockSpec((NB, BC, NHEAD), lambda i: (0, i, 0)),
            full(DA, DA),
            full(DKV, NHEAD),
            full(NHEAD, DKV),
            full(DKV, DA),
            full(DA, DA),
            full(DA, AN * DFF),
            full(1, AN * DFF),
            full(AN * DFF, DA),
            full(1, DA),
            full(1, DA),
            full(1, DA),
            full(DA, DA),
            full(1, DA),
        ],
        out_specs=pl.BlockSpec((BC, DA), lambda i: (i, 0)),
        out_shape=jax.ShapeDtypeStruct((N, DA), jnp.float32),
    )(tgt, tq, g3, dist_t, p_mat, h8, h8t, wo, ma, w1k, b1k, w2k, b2k,
      g1t, be1t, g2p, be2d)


def kernel(tgt, index_pair, cnt, sh, dist_atten, Wq, Wk, Wout, W1, b1, W2,
           b2, g1, be1, g2, be2):
    del cnt, sh
    f32 = jnp.float32
    eye5 = jnp.eye(AN, dtype=f32)

    def expand_in(w):
        m = w.T[:, None, None, :] * eye5[None, :, :, None]
        return m.reshape(DA, DA)

    wkt = expand_in(Wk)
    wqt = expand_in(Wq)
    zda = jnp.zeros((DA, DA), f32)
    wkv = jnp.concatenate([wkt, wqt], axis=1)
    wqq = jnp.concatenate([wqt, zda], axis=1)
    p_mat = expand_in(jnp.eye(D_MODEL, dtype=f32))
    wo = jnp.kron(eye5, Wout.T)
    wo640 = jnp.concatenate([zda, wo], axis=0)
    ma = jnp.kron(eye5, jnp.full((D_MODEL, D_MODEL), 1.0 / D_MODEL, f32))
    w1k = jnp.kron(eye5, W1.T)
    w2k = jnp.kron(eye5, W2.T)
    b1k = jnp.tile(b1, AN)[None, :]
    b2k = jnp.tile(b2, AN)[None, :]
    g1t = jnp.tile(g1, AN)[None, :]
    be1t = jnp.tile(be1, AN)[None, :]
    g2t = jnp.tile(g2, AN)
    be2t = jnp.tile(be2, AN)
    scale = 1.0 / (DPH ** 0.5)
    dmaj = jnp.tile(jnp.arange(D_MODEL), AN)
    h8 = (dmaj[:, None] // DPH == jnp.arange(NHEAD)[None, :]).astype(f32)
    h8pad = jnp.concatenate([h8 * scale, jnp.zeros((DA, NHEAD), f32)], axis=0)
    h8t = jnp.concatenate([jnp.zeros((NHEAD, DA), f32), h8.T], axis=1)
    g2p = g2t[:, None] * p_mat.T
    be2d = be2t[None, :] @ p_mat.T

    kv, tq = _project(tgt, wkv, wqq)
    idx_t = index_pair.astype(jnp.int32).T.reshape(-1)
    g = _gather(kv, idx_t)
    g3 = g.reshape(NB, N, DKV)
    dist_t = jnp.transpose(dist_atten, (1, 0, 2))
    return _attn_tail(tgt, tq, g3, dist_t, p_mat, h8pad, h8t, wo640, ma,
                      w1k, b1k, w2k, b2k, g1t, be1t, g2p, be2d)
